# Initial kernel scaffold; baseline (speedup 1.0000x reference)
import jax
import jax.numpy as jnp
from jax.experimental import pallas as pl


def kernel(x, Wg, bg, W1, b1, W2, b2):
    raise NotImplementedError("write your pallas kernel here")


# speedup vs baseline: 1.8701x; 1.8701x over previous
import functools

import jax
import jax.numpy as jnp
from jax import lax
from jax.experimental import pallas as pl
from jax.experimental.pallas import tpu as pltpu
from jax.experimental.pallas import tpu_sc as plsc

E = 16
D = 1024
F = 4096
N = 4096
TB = 256
NB = N // TB
TP = 256
PMAX = 2 * N + E * TP
NT = PMAX // TP
NTE = 64
L = 16
NC = 2
NS = 16
NW = NC * NS
QW = PMAX // NW
TW = N // NW
GC = 64
GT = 32


def _gate_body(x_ref, wg_ref, bg_ref,
               p0_ref, p1_ref, e0_ref, e1_ref, r0_ref, r1_ref,
               starts_ref, te_ref, aux_ref,
               carry, tpe, psum):
    b = pl.program_id(0)

    @pl.when(b == 0)
    def _init():
        carry[...] = jnp.zeros_like(carry)
        tpe[...] = jnp.zeros_like(tpe)
        psum[...] = jnp.zeros_like(psum)

    xb = x_ref[...]
    logits = jnp.dot(xb, wg_ref[...], preferred_element_type=jnp.float32)
    logits = logits + bg_ref[...]
    m = jnp.max(logits, axis=1, keepdims=True)
    ex = jnp.exp(logits - m)
    probs = ex / jnp.sum(ex, axis=1, keepdims=True)

    iota = lax.broadcasted_iota(jnp.int32, (TB, E), 1)
    m1 = jnp.max(probs, axis=1, keepdims=True)
    a1 = jnp.min(jnp.where(probs == m1, iota, E), axis=1, keepdims=True)
    p0sel = iota == a1
    masked = jnp.where(p0sel, -jnp.inf, probs)
    m2 = jnp.max(masked, axis=1, keepdims=True)
    a2 = jnp.min(jnp.where(masked == m2, iota, E), axis=1, keepdims=True)
    p1sel = iota == a2

    P0 = p0sel.astype(jnp.float32)
    P1 = p1sel.astype(jnp.float32)
    S = P0 + P1
    ti = lax.broadcasted_iota(jnp.int32, (TB, TB), 0)
    tj = lax.broadcasted_iota(jnp.int32, (TB, TB), 1)
    tril = (ti > tj).astype(jnp.float32)
    excl = jnp.dot(tril, S, preferred_element_type=jnp.float32)
    base = carry[...] + excl
    r0 = jnp.sum(base * P0, axis=1)
    r1 = jnp.sum(base * P1, axis=1)

    p0_ref[...] = m1[:, 0].reshape(1, 1, TB)
    p1_ref[...] = m2[:, 0].reshape(1, 1, TB)
    e0_ref[...] = a1[:, 0].reshape(1, 1, TB)
    e1_ref[...] = a2[:, 0].reshape(1, 1, TB)
    r0_ref[...] = r0.astype(jnp.int32).reshape(1, 1, TB)
    r1_ref[...] = r1.astype(jnp.int32).reshape(1, 1, TB)

    carry[...] = carry[...] + jnp.sum(S, axis=0, keepdims=True)
    tpe[...] = tpe[...] + jnp.sum(P0, axis=0, keepdims=True)
    psum[...] = psum[...] + jnp.sum(probs, axis=0, keepdims=True)

    padded = jnp.floor((carry[...] + (TP - 1)) / TP) * TP
    ei = lax.broadcasted_iota(jnp.int32, (E, E), 0)
    ej = lax.broadcasted_iota(jnp.int32, (E, E), 1)
    incl = (ei <= ej).astype(jnp.float32)
    ends = jnp.dot(padded, incl, preferred_element_type=jnp.float32)
    starts_ref[...] = (ends - padded).astype(jnp.int32)
    gv = lax.broadcasted_iota(jnp.int32, (NTE, 1), 0).astype(jnp.float32) * TP
    te = jnp.sum((gv >= ends).astype(jnp.int32), axis=1)
    te_ref[...] = jnp.minimum(te, E - 1).reshape(1, NTE)
    aux_ref[...] = (jnp.sum(tpe[...] / (N + 1e-8) * (psum[...] / N)) * E
                    ).reshape(1, 1)


def _gate_call(x, Wg, bg2):
    f32 = jnp.float32
    i32 = jnp.int32
    pair_shape = jax.ShapeDtypeStruct((NB, 1, TB), f32)
    pair_ishape = jax.ShapeDtypeStruct((NB, 1, TB), i32)
    pair_spec = pl.BlockSpec((1, 1, TB), lambda i: (i, 0, 0))
    one_spec = lambda s: pl.BlockSpec(s, lambda i: tuple(0 for _ in s))
    return pl.pallas_call(
        _gate_body,
        grid=(NB,),
        in_specs=[
            pl.BlockSpec((TB, D), lambda i: (i, 0)),
            one_spec((D, E)),
            one_spec((1, E)),
        ],
        out_specs=[pair_spec, pair_spec, pair_spec, pair_spec, pair_spec,
                   pair_spec, one_spec((1, E)), one_spec((1, NTE)),
                   one_spec((1, 1))],
        out_shape=[pair_shape, pair_shape, pair_ishape, pair_ishape,
                   pair_ishape, pair_ishape,
                   jax.ShapeDtypeStruct((1, E), i32),
                   jax.ShapeDtypeStruct((1, NTE), i32),
                   jax.ShapeDtypeStruct((1, 1), f32)],
        scratch_shapes=[pltpu.VMEM((1, E), f32)] * 3,
    )(x, Wg, bg2)


def _route_body(starts_hbm, e0_hbm, e1_hbm, r0_hbm, r1_hbm, p0_hbm, p1_hbm,
                x_hbm,
                xs_hbm, ws_hbm, d0_hbm, d1_hbm,
                starts_v, e0_v, e1_v, r0_v, r1_v, p0_v, p1_v,
                src_v, wloc_v, d0loc_v, d1loc_v, rows_v, sem):
    wid = lax.axis_index("s") * NC + lax.axis_index("c")
    qw = wid * QW
    t0 = wid * TW

    pltpu.sync_copy(starts_hbm, starts_v)
    pltpu.sync_copy(e0_hbm, e0_v)
    pltpu.sync_copy(e1_hbm, e1_v)
    pltpu.sync_copy(r0_hbm, r0_v)
    pltpu.sync_copy(r1_hbm, r1_v)
    pltpu.sync_copy(p0_hbm, p0_v)
    pltpu.sync_copy(p1_hbm, p1_v)

    zl_i = jnp.zeros((L,), jnp.int32)
    zl_f = jnp.zeros((L,), jnp.float32)
    for k in range(QW // L):
        src_v[pl.ds(k * L, L)] = zl_i
        wloc_v[pl.ds(k * L, L)] = zl_f

    lanes = lax.iota(jnp.int32, L)

    def scan_body(i, _):
        sl = pl.ds(i * L, L)
        tok = i * L + lanes
        e0c = e0_v[sl]
        e1c = e1_v[sl]
        d0c = plsc.load_gather(starts_v, [e0c]) + r0_v[sl]
        d1c = plsc.load_gather(starts_v, [e1c]) + r1_v[sl]
        m0 = (d0c >= qw) & (d0c < qw + QW)
        m1_ = (d1c >= qw) & (d1c < qw + QW)
        i0 = jnp.where(m0, d0c - qw, 0)
        i1 = jnp.where(m1_, d1c - qw, 0)
        plsc.store_scatter(src_v, [i0], tok, mask=m0)
        plsc.store_scatter(wloc_v, [i0], p0_v[sl], mask=m0)
        plsc.store_scatter(src_v, [i1], tok, mask=m1_)
        plsc.store_scatter(wloc_v, [i1], p1_v[sl], mask=m1_)
        return 0

    lax.fori_loop(0, N // L, scan_body, 0)

    for k in range(TW // L):
        sl = pl.ds(t0 + k * L, L)
        osl = pl.ds(k * L, L)
        d0loc_v[osl] = plsc.load_gather(starts_v, [e0_v[sl]]) + r0_v[sl]
        d1loc_v[osl] = plsc.load_gather(starts_v, [e1_v[sl]]) + r1_v[sl]
    pltpu.sync_copy(d0loc_v, d0_hbm.at[pl.ds(t0, TW)])
    pltpu.sync_copy(d1loc_v, d1_hbm.at[pl.ds(t0, TW)])
    pltpu.sync_copy(wloc_v, ws_hbm.at[pl.ds(qw, QW)])

    for k in range(QW // GC):
        idx = src_v.at[pl.ds(k * GC, GC)]
        pltpu.async_copy(x_hbm.at[idx], rows_v, sem).wait()
        pltpu.sync_copy(rows_v, xs_hbm.at[pl.ds(qw + k * GC, GC)])


def _route_call(starts, e0, e1, r0, r1, p0, p1, x):
    f32 = jnp.float32
    i32 = jnp.int32
    mesh = plsc.VectorSubcoreMesh(core_axis_name="c", subcore_axis_name="s",
                                  num_cores=NC, num_subcores=NS)
    return pl.kernel(
        _route_body,
        out_type=[jax.ShapeDtypeStruct((PMAX, D), f32),
                  jax.ShapeDtypeStruct((PMAX,), f32),
                  jax.ShapeDtypeStruct((N,), i32),
                  jax.ShapeDtypeStruct((N,), i32)],
        mesh=mesh,
        compiler_params=pltpu.CompilerParams(needs_layout_passes=False),
        scratch_types=[
            pltpu.VMEM((E,), i32),
            pltpu.VMEM((N,), i32), pltpu.VMEM((N,), i32),
            pltpu.VMEM((N,), i32), pltpu.VMEM((N,), i32),
            pltpu.VMEM((N,), f32), pltpu.VMEM((N,), f32),
            pltpu.VMEM((QW,), i32),
            pltpu.VMEM((QW,), f32),
            pltpu.VMEM((TW,), i32), pltpu.VMEM((TW,), i32),
            pltpu.VMEM((GC, D), f32),
            pltpu.SemaphoreType.DMA,
        ],
    )(starts, e0, e1, r0, r1, p0, p1, x)


def _ffn_body(te_ref, xs_ref, w1_ref, b1_ref, w2_ref, b2_ref, ws_ref,
              ys_ref):
    xb = xs_ref[...].astype(jnp.bfloat16)
    h = jnp.dot(xb, w1_ref[0], preferred_element_type=jnp.float32)
    h = jnp.maximum(h + b1_ref[0], 0.0)
    y = jnp.dot(h.astype(jnp.bfloat16), w2_ref[0],
                preferred_element_type=jnp.float32)
    y = y + b2_ref[0]
    ys_ref[...] = y * ws_ref[0, 0][:, None]


def _ffn_call(te, xs, W1b, b1, W2b, b2, ws3):
    grid_spec = pltpu.PrefetchScalarGridSpec(
        num_scalar_prefetch=1,
        grid=(NT,),
        in_specs=[
            pl.BlockSpec((TP, D), lambda i, te: (i, 0)),
            pl.BlockSpec((1, D, F), lambda i, te: (te[i], 0, 0)),
            pl.BlockSpec((1, 1, F), lambda i, te: (te[i], 0, 0)),
            pl.BlockSpec((1, F, D), lambda i, te: (te[i], 0, 0)),
            pl.BlockSpec((1, 1, D), lambda i, te: (te[i], 0, 0)),
            pl.BlockSpec((1, 1, TP), lambda i, te: (i, 0, 0)),
        ],
        out_specs=pl.BlockSpec((TP, D), lambda i, te: (i, 0)),
    )
    return pl.pallas_call(
        _ffn_body,
        grid_spec=grid_spec,
        out_shape=jax.ShapeDtypeStruct((PMAX, D), jnp.float32),
        compiler_params=pltpu.CompilerParams(
            dimension_semantics=("arbitrary",)),
    )(te, xs, W1b, b1, W2b, b2, ws3)


def _combine_body(ys_hbm, d0_hbm, d1_hbm, out_hbm,
                  d0_v, d1_v, a_v, b_v, o_v, sem):
    wid = lax.axis_index("s") * NC + lax.axis_index("c")
    t0 = wid * TW
    pltpu.sync_copy(d0_hbm.at[pl.ds(t0, TW)], d0_v)
    pltpu.sync_copy(d1_hbm.at[pl.ds(t0, TW)], d1_v)

    for k in range(TW // GT):
        pltpu.async_copy(ys_hbm.at[d0_v.at[pl.ds(k * GT, GT)]], a_v,
                         sem).wait()
        pltpu.async_copy(ys_hbm.at[d1_v.at[pl.ds(k * GT, GT)]], b_v,
                         sem).wait()

        def add_body(t, _):
            for cch in range(D // L):
                sl = pl.ds(cch * L, L)
                o_v[t, sl] = a_v[t, sl] + b_v[t, sl]
            return 0

        lax.fori_loop(0, GT, add_body, 0)
        pltpu.sync_copy(o_v, out_hbm.at[pl.ds(t0 + k * GT, GT)])


def _combine_call(ys, d0, d1):
    f32 = jnp.float32
    i32 = jnp.int32
    mesh = plsc.VectorSubcoreMesh(core_axis_name="c", subcore_axis_name="s",
                                  num_cores=NC, num_subcores=NS)
    return pl.kernel(
        _combine_body,
        out_type=jax.ShapeDtypeStruct((N, D), f32),
        mesh=mesh,
        compiler_params=pltpu.CompilerParams(needs_layout_passes=False),
        scratch_types=[
            pltpu.VMEM((TW,), i32), pltpu.VMEM((TW,), i32),
            pltpu.VMEM((GT, D), f32), pltpu.VMEM((GT, D), f32),
            pltpu.VMEM((GT, D), f32),
            pltpu.SemaphoreType.DMA,
        ],
    )(ys, d0, d1)


def kernel(x, Wg, bg, W1, b1, W2, b2):
    p0, p1, e0, e1, r0, r1, starts, te, aux = _gate_call(x, Wg,
                                                         bg.reshape(1, E))
    flat = lambda a: a.reshape(-1)
    xs, ws, d0, d1 = _route_call(starts.reshape(E), flat(e0), flat(e1),
                                 flat(r0), flat(r1), flat(p0), flat(p1), x)
    ys = _ffn_call(te.reshape(NTE)[:NT], xs,
                   W1.astype(jnp.bfloat16), b1.reshape(E, 1, F),
                   W2.astype(jnp.bfloat16), b2.reshape(E, 1, D),
                   ws.reshape(NT, 1, TP))
    out = _combine_call(ys, d0, d1)
    return out, aux.reshape(())


# speedup vs baseline: 2.5746x; 1.3767x over previous
import functools

import jax
import jax.numpy as jnp
from jax import lax
from jax.experimental import pallas as pl
from jax.experimental.pallas import tpu as pltpu
from jax.experimental.pallas import tpu_sc as plsc

E = 16
D = 1024
F = 4096
N = 4096
TB = 256
NB = N // TB
TP = 256
PMAX = 2 * N + E * TP
NT = PMAX // TP
NTE = 64
L = 16
NC = 2
NS = 16
NW = NC * NS
QW = PMAX // NW
TW = N // NW
GC = 64
GT = 32


def _gate_body(x_ref, wg_ref, bg_ref,
               p0_ref, p1_ref, e0_ref, e1_ref, r0_ref, r1_ref,
               starts_ref, te_ref, aux_ref,
               carry, tpe, psum):
    b = pl.program_id(0)

    @pl.when(b == 0)
    def _init():
        carry[...] = jnp.zeros_like(carry)
        tpe[...] = jnp.zeros_like(tpe)
        psum[...] = jnp.zeros_like(psum)

    xb = x_ref[...]
    logits = jnp.dot(xb, wg_ref[...], preferred_element_type=jnp.float32)
    logits = logits + bg_ref[...]
    m = jnp.max(logits, axis=1, keepdims=True)
    ex = jnp.exp(logits - m)
    probs = ex / jnp.sum(ex, axis=1, keepdims=True)

    iota = lax.broadcasted_iota(jnp.int32, (TB, E), 1)
    m1 = jnp.max(probs, axis=1, keepdims=True)
    a1 = jnp.min(jnp.where(probs == m1, iota, E), axis=1, keepdims=True)
    p0sel = iota == a1
    masked = jnp.where(p0sel, -jnp.inf, probs)
    m2 = jnp.max(masked, axis=1, keepdims=True)
    a2 = jnp.min(jnp.where(masked == m2, iota, E), axis=1, keepdims=True)
    p1sel = iota == a2

    P0 = p0sel.astype(jnp.float32)
    P1 = p1sel.astype(jnp.float32)
    S = P0 + P1
    ti = lax.broadcasted_iota(jnp.int32, (TB, TB), 0)
    tj = lax.broadcasted_iota(jnp.int32, (TB, TB), 1)
    tril = (ti > tj).astype(jnp.float32)
    excl = jnp.dot(tril, S, preferred_element_type=jnp.float32)
    base = carry[...] + excl
    r0 = jnp.sum(base * P0, axis=1)
    r1 = jnp.sum(base * P1, axis=1)

    p0_ref[...] = m1[:, 0].reshape(1, 1, TB)
    p1_ref[...] = m2[:, 0].reshape(1, 1, TB)
    e0_ref[...] = a1[:, 0].reshape(1, 1, TB)
    e1_ref[...] = a2[:, 0].reshape(1, 1, TB)
    r0_ref[...] = r0.astype(jnp.int32).reshape(1, 1, TB)
    r1_ref[...] = r1.astype(jnp.int32).reshape(1, 1, TB)

    carry[...] = carry[...] + jnp.sum(S, axis=0, keepdims=True)
    tpe[...] = tpe[...] + jnp.sum(P0, axis=0, keepdims=True)
    psum[...] = psum[...] + jnp.sum(probs, axis=0, keepdims=True)

    padded = jnp.floor((carry[...] + (TP - 1)) / TP) * TP
    ei = lax.broadcasted_iota(jnp.int32, (E, E), 0)
    ej = lax.broadcasted_iota(jnp.int32, (E, E), 1)
    incl = (ei <= ej).astype(jnp.float32)
    ends = jnp.dot(padded, incl, preferred_element_type=jnp.float32)
    starts_ref[...] = (ends - padded).astype(jnp.int32)
    gv = lax.broadcasted_iota(jnp.int32, (NTE, 1), 0).astype(jnp.float32) * TP
    te = jnp.sum((gv >= ends).astype(jnp.int32), axis=1)
    te_ref[...] = jnp.minimum(te, E - 1).reshape(1, NTE)
    aux_ref[...] = (jnp.sum(tpe[...] / (N + 1e-8) * (psum[...] / N)) * E
                    ).reshape(1, 1)


def _gate_call(x, Wg, bg2):
    f32 = jnp.float32
    i32 = jnp.int32
    pair_shape = jax.ShapeDtypeStruct((NB, 1, TB), f32)
    pair_ishape = jax.ShapeDtypeStruct((NB, 1, TB), i32)
    pair_spec = pl.BlockSpec((1, 1, TB), lambda i: (i, 0, 0))
    one_spec = lambda s: pl.BlockSpec(s, lambda i: tuple(0 for _ in s))
    return pl.pallas_call(
        _gate_body,
        grid=(NB,),
        in_specs=[
            pl.BlockSpec((TB, D), lambda i: (i, 0)),
            one_spec((D, E)),
            one_spec((1, E)),
        ],
        out_specs=[pair_spec, pair_spec, pair_spec, pair_spec, pair_spec,
                   pair_spec, one_spec((1, E)), one_spec((1, NTE)),
                   one_spec((1, 1))],
        out_shape=[pair_shape, pair_shape, pair_ishape, pair_ishape,
                   pair_ishape, pair_ishape,
                   jax.ShapeDtypeStruct((1, E), i32),
                   jax.ShapeDtypeStruct((1, NTE), i32),
                   jax.ShapeDtypeStruct((1, 1), f32)],
        scratch_shapes=[pltpu.VMEM((1, E), f32)] * 3,
    )(x, Wg, bg2)


KCH = TW // GC


def _route_body(starts_hbm, e0_hbm, e1_hbm, r0_hbm, r1_hbm, x_hbm,
                xs_hbm, d0_hbm, d1_hbm,
                starts_v, e0_v, e1_v, r0_v, r1_v,
                d0f_v, d1f_v, d0i_v, d1i_v, rows_v, sem):
    wid = lax.axis_index("s") * NC + lax.axis_index("c")
    t0 = wid * TW

    pltpu.sync_copy(starts_hbm, starts_v)
    pltpu.sync_copy(e0_hbm.at[pl.ds(t0, TW)], e0_v)
    pltpu.sync_copy(e1_hbm.at[pl.ds(t0, TW)], e1_v)
    pltpu.sync_copy(r0_hbm.at[pl.ds(t0, TW)], r0_v)
    pltpu.sync_copy(r1_hbm.at[pl.ds(t0, TW)], r1_v)

    for k in range(TW // L):
        sl = pl.ds(k * L, L)
        d0c = plsc.load_gather(starts_v, [e0_v[sl]]) + r0_v[sl]
        d1c = plsc.load_gather(starts_v, [e1_v[sl]]) + r1_v[sl]
        d0f_v[sl] = d0c
        d1f_v[sl] = d1c
        d0i_v[k // 4, pl.ds((k % 4) * L, L)] = d0c
        d1i_v[k // 4, pl.ds((k % 4) * L, L)] = d1c
    pltpu.sync_copy(d0f_v, d0_hbm.at[pl.ds(t0, TW)])
    pltpu.sync_copy(d1f_v, d1_hbm.at[pl.ds(t0, TW)])

    for c in range(KCH):
        pltpu.sync_copy(x_hbm.at[pl.ds(t0 + c * GC, GC)], rows_v)
        cp0 = pltpu.async_copy(rows_v, xs_hbm.at[d0i_v.at[c]], sem)
        cp1 = pltpu.async_copy(rows_v, xs_hbm.at[d1i_v.at[c]], sem)
        cp0.wait()
        cp1.wait()


def _route_call(starts, e0, e1, r0, r1, x):
    f32 = jnp.float32
    i32 = jnp.int32
    mesh = plsc.VectorSubcoreMesh(core_axis_name="c", subcore_axis_name="s",
                                  num_cores=NC, num_subcores=NS)
    return pl.kernel(
        _route_body,
        out_type=[jax.ShapeDtypeStruct((PMAX, D), f32),
                  jax.ShapeDtypeStruct((N,), i32),
                  jax.ShapeDtypeStruct((N,), i32)],
        mesh=mesh,
        compiler_params=pltpu.CompilerParams(needs_layout_passes=False),
        scratch_types=[
            pltpu.VMEM((E,), i32),
            pltpu.VMEM((TW,), i32), pltpu.VMEM((TW,), i32),
            pltpu.VMEM((TW,), i32), pltpu.VMEM((TW,), i32),
            pltpu.VMEM((TW,), i32), pltpu.VMEM((TW,), i32),
            pltpu.VMEM((KCH, GC), i32), pltpu.VMEM((KCH, GC), i32),
            pltpu.VMEM((GC, D), f32),
            pltpu.SemaphoreType.DMA,
        ],
    )(starts, e0, e1, r0, r1, x)


def _ffn_body(te_ref, xs_ref, w1_ref, b1_ref, w2_ref, b2_ref, ys_ref):
    xb = xs_ref[...].astype(jnp.bfloat16)
    h = jnp.dot(xb, w1_ref[0], preferred_element_type=jnp.float32)
    h = jnp.maximum(h + b1_ref[0], 0.0)
    y = jnp.dot(h.astype(jnp.bfloat16), w2_ref[0],
                preferred_element_type=jnp.float32)
    ys_ref[...] = y + b2_ref[0]


def _ffn_call(te, xs, W1b, b1, W2b, b2):
    grid_spec = pltpu.PrefetchScalarGridSpec(
        num_scalar_prefetch=1,
        grid=(NT,),
        in_specs=[
            pl.BlockSpec((TP, D), lambda i, te: (i, 0)),
            pl.BlockSpec((1, D, F), lambda i, te: (te[i], 0, 0)),
            pl.BlockSpec((1, 1, F), lambda i, te: (te[i], 0, 0)),
            pl.BlockSpec((1, F, D), lambda i, te: (te[i], 0, 0)),
            pl.BlockSpec((1, 1, D), lambda i, te: (te[i], 0, 0)),
        ],
        out_specs=pl.BlockSpec((TP, D), lambda i, te: (i, 0)),
    )
    return pl.pallas_call(
        _ffn_body,
        grid_spec=grid_spec,
        out_shape=jax.ShapeDtypeStruct((PMAX, D), jnp.float32),
        compiler_params=pltpu.CompilerParams(
            dimension_semantics=("arbitrary",)),
    )(te, xs, W1b, b1, W2b, b2)


def _combine_body(ys_hbm, d0_hbm, d1_hbm, p0_hbm, p1_hbm, out_hbm,
                  d0_v, d1_v, p0_v, p1_v, a_v, b_v, o_v, sem):
    wid = lax.axis_index("s") * NC + lax.axis_index("c")
    t0 = wid * TW
    pltpu.sync_copy(d0_hbm.at[pl.ds(t0, TW)], d0_v)
    pltpu.sync_copy(d1_hbm.at[pl.ds(t0, TW)], d1_v)
    pltpu.sync_copy(p0_hbm.at[pl.ds(t0, TW)], p0_v)
    pltpu.sync_copy(p1_hbm.at[pl.ds(t0, TW)], p1_v)

    lanes = lax.iota(jnp.int32, L)
    zl = jnp.zeros((L,), jnp.float32)

    for k in range(TW // GT):
        pltpu.async_copy(ys_hbm.at[d0_v.at[pl.ds(k * GT, GT)]], a_v,
                         sem).wait()
        pltpu.async_copy(ys_hbm.at[d1_v.at[pl.ds(k * GT, GT)]], b_v,
                         sem).wait()

        def comb_body(t, _):
            g = k * GT + t
            base = jnp.bitwise_and(g, ~(L - 1))
            lane = jnp.bitwise_and(g, L - 1)
            pv0 = p0_v[pl.ds(base, L)]
            pv1 = p1_v[pl.ds(base, L)]
            w0 = jnp.sum(jnp.where(lanes == lane, pv0, zl))
            w1 = jnp.sum(jnp.where(lanes == lane, pv1, zl))
            for cch in range(D // L):
                sl = pl.ds(cch * L, L)
                o_v[t, sl] = w0 * a_v[t, sl] + w1 * b_v[t, sl]
            return 0

        lax.fori_loop(0, GT, comb_body, 0)
        pltpu.sync_copy(o_v, out_hbm.at[pl.ds(t0 + k * GT, GT)])


def _combine_call(ys, d0, d1, p0, p1):
    f32 = jnp.float32
    i32 = jnp.int32
    mesh = plsc.VectorSubcoreMesh(core_axis_name="c", subcore_axis_name="s",
                                  num_cores=NC, num_subcores=NS)
    return pl.kernel(
        _combine_body,
        out_type=jax.ShapeDtypeStruct((N, D), f32),
        mesh=mesh,
        compiler_params=pltpu.CompilerParams(needs_layout_passes=False),
        scratch_types=[
            pltpu.VMEM((TW,), i32), pltpu.VMEM((TW,), i32),
            pltpu.VMEM((TW,), f32), pltpu.VMEM((TW,), f32),
            pltpu.VMEM((GT, D), f32), pltpu.VMEM((GT, D), f32),
            pltpu.VMEM((GT, D), f32),
            pltpu.SemaphoreType.DMA,
        ],
    )(ys, d0, d1, p0, p1)


def kernel(x, Wg, bg, W1, b1, W2, b2):
    p0, p1, e0, e1, r0, r1, starts, te, aux = _gate_call(x, Wg,
                                                         bg.reshape(1, E))
    flat = lambda a: a.reshape(-1)
    xs, d0, d1 = _route_call(starts.reshape(E), flat(e0), flat(e1),
                             flat(r0), flat(r1), x)
    ys = _ffn_call(te.reshape(NTE)[:NT], xs,
                   W1.astype(jnp.bfloat16), b1.reshape(E, 1, F),
                   W2.astype(jnp.bfloat16), b2.reshape(E, 1, D))
    out = _combine_call(ys, d0, d1, flat(p0), flat(p1))
    return out, aux.reshape(())


# speedup vs baseline: 3.0911x; 1.2006x over previous
import functools

import jax
import jax.numpy as jnp
from jax import lax
from jax.experimental import pallas as pl
from jax.experimental.pallas import tpu as pltpu
from jax.experimental.pallas import tpu_sc as plsc

E = 16
D = 1024
F = 4096
N = 4096
TB = 512
NB = N // TB
TP = 256
PMAX = 2 * N + E * TP
NT = PMAX // TP
NTE = 64
L = 16
NC = 2
NS = 16
NW = NC * NS
QW = PMAX // NW
TW = N // NW
GC = 64
GT = 32


def _gate_body(x_ref, wg_ref, bg_ref,
               p0_ref, p1_ref, e0_ref, e1_ref, r0_ref, r1_ref,
               starts_ref, te_ref, aux_ref,
               carry, tpe, psum):
    b = pl.program_id(0)

    @pl.when(b == 0)
    def _init():
        carry[...] = jnp.zeros_like(carry)
        tpe[...] = jnp.zeros_like(tpe)
        psum[...] = jnp.zeros_like(psum)

    xb = x_ref[...]
    logits = jnp.dot(xb, wg_ref[...], preferred_element_type=jnp.float32)
    logits = logits + bg_ref[...]
    m = jnp.max(logits, axis=1, keepdims=True)
    ex = jnp.exp(logits - m)
    probs = ex / jnp.sum(ex, axis=1, keepdims=True)

    iota = lax.broadcasted_iota(jnp.int32, (TB, E), 1)
    m1 = jnp.max(probs, axis=1, keepdims=True)
    a1 = jnp.min(jnp.where(probs == m1, iota, E), axis=1, keepdims=True)
    p0sel = iota == a1
    masked = jnp.where(p0sel, -jnp.inf, probs)
    m2 = jnp.max(masked, axis=1, keepdims=True)
    a2 = jnp.min(jnp.where(masked == m2, iota, E), axis=1, keepdims=True)
    p1sel = iota == a2

    P0 = p0sel.astype(jnp.float32)
    P1 = p1sel.astype(jnp.float32)
    S = P0 + P1
    ti = lax.broadcasted_iota(jnp.int32, (TB, TB), 0)
    tj = lax.broadcasted_iota(jnp.int32, (TB, TB), 1)
    tril = (ti > tj).astype(jnp.float32)
    excl = jnp.dot(tril, S, preferred_element_type=jnp.float32)
    base = carry[...] + excl
    r0 = jnp.sum(base * P0, axis=1)
    r1 = jnp.sum(base * P1, axis=1)

    p0_ref[...] = m1[:, 0].reshape(1, 1, TB)
    p1_ref[...] = m2[:, 0].reshape(1, 1, TB)
    e0_ref[...] = a1[:, 0].reshape(1, 1, TB)
    e1_ref[...] = a2[:, 0].reshape(1, 1, TB)
    r0_ref[...] = r0.astype(jnp.int32).reshape(1, 1, TB)
    r1_ref[...] = r1.astype(jnp.int32).reshape(1, 1, TB)

    carry[...] = carry[...] + jnp.sum(S, axis=0, keepdims=True)
    tpe[...] = tpe[...] + jnp.sum(P0, axis=0, keepdims=True)
    psum[...] = psum[...] + jnp.sum(probs, axis=0, keepdims=True)

    padded = jnp.floor((carry[...] + (TP - 1)) / TP) * TP
    ei = lax.broadcasted_iota(jnp.int32, (E, E), 0)
    ej = lax.broadcasted_iota(jnp.int32, (E, E), 1)
    incl = (ei <= ej).astype(jnp.float32)
    ends = jnp.dot(padded, incl, preferred_element_type=jnp.float32)
    starts_ref[...] = (ends - padded).astype(jnp.int32)
    gv = lax.broadcasted_iota(jnp.int32, (NTE, 1), 0).astype(jnp.float32) * TP
    te = jnp.sum((gv >= ends).astype(jnp.int32), axis=1)
    te_ref[...] = jnp.minimum(te, E - 1).reshape(1, NTE)
    aux_ref[...] = (jnp.sum(tpe[...] / (N + 1e-8) * (psum[...] / N)) * E
                    ).reshape(1, 1)


def _gate_call(x, Wg, bg2):
    f32 = jnp.float32
    i32 = jnp.int32
    pair_shape = jax.ShapeDtypeStruct((NB, 1, TB), f32)
    pair_ishape = jax.ShapeDtypeStruct((NB, 1, TB), i32)
    pair_spec = pl.BlockSpec((1, 1, TB), lambda i: (i, 0, 0))
    one_spec = lambda s: pl.BlockSpec(s, lambda i: tuple(0 for _ in s))
    return pl.pallas_call(
        _gate_body,
        grid=(NB,),
        in_specs=[
            pl.BlockSpec((TB, D), lambda i: (i, 0)),
            one_spec((D, E)),
            one_spec((1, E)),
        ],
        out_specs=[pair_spec, pair_spec, pair_spec, pair_spec, pair_spec,
                   pair_spec, one_spec((1, E)), one_spec((1, NTE)),
                   one_spec((1, 1))],
        out_shape=[pair_shape, pair_shape, pair_ishape, pair_ishape,
                   pair_ishape, pair_ishape,
                   jax.ShapeDtypeStruct((1, E), i32),
                   jax.ShapeDtypeStruct((1, NTE), i32),
                   jax.ShapeDtypeStruct((1, 1), f32)],
        scratch_shapes=[pltpu.VMEM((1, E), f32)] * 3,
    )(x, Wg, bg2)


KCH = TW // GC


def _route_body(starts_hbm, e0_hbm, e1_hbm, r0_hbm, r1_hbm, x_hbm,
                xs_hbm, d0_hbm, d1_hbm,
                starts_v, e0_v, e1_v, r0_v, r1_v,
                d0f_v, d1f_v, d0i_v, d1i_v, rows_v, sem):
    wid = lax.axis_index("s") * NC + lax.axis_index("c")
    t0 = wid * TW

    pltpu.sync_copy(starts_hbm, starts_v)
    pltpu.sync_copy(e0_hbm.at[pl.ds(t0, TW)], e0_v)
    pltpu.sync_copy(e1_hbm.at[pl.ds(t0, TW)], e1_v)
    pltpu.sync_copy(r0_hbm.at[pl.ds(t0, TW)], r0_v)
    pltpu.sync_copy(r1_hbm.at[pl.ds(t0, TW)], r1_v)

    for k in range(TW // L):
        sl = pl.ds(k * L, L)
        d0c = plsc.load_gather(starts_v, [e0_v[sl]]) + r0_v[sl]
        d1c = plsc.load_gather(starts_v, [e1_v[sl]]) + r1_v[sl]
        d0f_v[sl] = d0c
        d1f_v[sl] = d1c
        d0i_v[k // 4, pl.ds((k % 4) * L, L)] = d0c
        d1i_v[k // 4, pl.ds((k % 4) * L, L)] = d1c
    pltpu.sync_copy(d0f_v, d0_hbm.at[pl.ds(t0, TW)])
    pltpu.sync_copy(d1f_v, d1_hbm.at[pl.ds(t0, TW)])

    for c in range(KCH):
        pltpu.sync_copy(x_hbm.at[pl.ds(t0 + c * GC, GC)], rows_v)
        cp0 = pltpu.async_copy(rows_v, xs_hbm.at[d0i_v.at[c]], sem)
        cp1 = pltpu.async_copy(rows_v, xs_hbm.at[d1i_v.at[c]], sem)
        cp0.wait()
        cp1.wait()


def _route_call(starts, e0, e1, r0, r1, x):
    f32 = jnp.float32
    i32 = jnp.int32
    mesh = plsc.VectorSubcoreMesh(core_axis_name="c", subcore_axis_name="s",
                                  num_cores=NC, num_subcores=NS)
    return pl.kernel(
        _route_body,
        out_type=[jax.ShapeDtypeStruct((PMAX, D), f32),
                  jax.ShapeDtypeStruct((N,), i32),
                  jax.ShapeDtypeStruct((N,), i32)],
        mesh=mesh,
        compiler_params=pltpu.CompilerParams(needs_layout_passes=False),
        scratch_types=[
            pltpu.VMEM((E,), i32),
            pltpu.VMEM((TW,), i32), pltpu.VMEM((TW,), i32),
            pltpu.VMEM((TW,), i32), pltpu.VMEM((TW,), i32),
            pltpu.VMEM((TW,), i32), pltpu.VMEM((TW,), i32),
            pltpu.VMEM((KCH, GC), i32), pltpu.VMEM((KCH, GC), i32),
            pltpu.VMEM((GC, D), f32),
            pltpu.SemaphoreType.DMA,
        ],
    )(starts, e0, e1, r0, r1, x)


def _ffn1_body(te_ref, xs_ref, w1_ref, b1_ref, h_ref):
    h = jnp.dot(xs_ref[...], w1_ref[0], preferred_element_type=jnp.float32)
    h_ref[...] = jnp.maximum(h + b1_ref[0], 0.0).astype(jnp.bfloat16)


def _ffn2_body(te_ref, h_ref, w2_ref, b2_ref, ys_ref):
    w2b = w2_ref[0].astype(jnp.bfloat16)
    y = jnp.dot(h_ref[...], w2b, preferred_element_type=jnp.float32)
    ys_ref[...] = y + b2_ref[0]


def _ffn_call(te, xs, W1, b1, W2, b2):
    g1 = pltpu.PrefetchScalarGridSpec(
        num_scalar_prefetch=1,
        grid=(NT,),
        in_specs=[
            pl.BlockSpec((TP, D), lambda i, te: (i, 0)),
            pl.BlockSpec((1, D, F), lambda i, te: (te[i], 0, 0)),
            pl.BlockSpec((1, 1, F), lambda i, te: (te[i], 0, 0)),
        ],
        out_specs=pl.BlockSpec((TP, F), lambda i, te: (i, 0)),
    )
    h = pl.pallas_call(
        _ffn1_body,
        grid_spec=g1,
        out_shape=jax.ShapeDtypeStruct((PMAX, F), jnp.bfloat16),
        compiler_params=pltpu.CompilerParams(
            dimension_semantics=("arbitrary",)),
    )(te, xs, W1, b1)
    g2 = pltpu.PrefetchScalarGridSpec(
        num_scalar_prefetch=1,
        grid=(NT,),
        in_specs=[
            pl.BlockSpec((TP, F), lambda i, te: (i, 0)),
            pl.BlockSpec((1, F, D), lambda i, te: (te[i], 0, 0)),
            pl.BlockSpec((1, 1, D), lambda i, te: (te[i], 0, 0)),
        ],
        out_specs=pl.BlockSpec((TP, D), lambda i, te: (i, 0)),
    )
    return pl.pallas_call(
        _ffn2_body,
        grid_spec=g2,
        out_shape=jax.ShapeDtypeStruct((PMAX, D), jnp.float32),
        compiler_params=pltpu.CompilerParams(
            dimension_semantics=("arbitrary",)),
    )(te, h, W2, b2)


def _combine_body(ys_hbm, d0_hbm, d1_hbm, p0_hbm, p1_hbm, out_hbm,
                  d0_v, d1_v, p0_v, p1_v, a_v, b_v, o_v, sem):
    wid = lax.axis_index("s") * NC + lax.axis_index("c")
    t0 = wid * TW
    pltpu.sync_copy(d0_hbm.at[pl.ds(t0, TW)], d0_v)
    pltpu.sync_copy(d1_hbm.at[pl.ds(t0, TW)], d1_v)
    pltpu.sync_copy(p0_hbm.at[pl.ds(t0, TW)], p0_v)
    pltpu.sync_copy(p1_hbm.at[pl.ds(t0, TW)], p1_v)

    lanes = lax.iota(jnp.int32, L)
    zl = jnp.zeros((L,), jnp.float32)

    for k in range(TW // GT):
        cpa = pltpu.async_copy(ys_hbm.at[d0_v.at[pl.ds(k * GT, GT)]], a_v,
                               sem)
        cpb = pltpu.async_copy(ys_hbm.at[d1_v.at[pl.ds(k * GT, GT)]], b_v,
                               sem)
        cpa.wait()
        cpb.wait()

        def comb_body(t, _):
            g = k * GT + t
            base = jnp.bitwise_and(g, ~(L - 1))
            lane = jnp.bitwise_and(g, L - 1)
            pv0 = p0_v[pl.ds(base, L)]
            pv1 = p1_v[pl.ds(base, L)]
            w0 = jnp.sum(jnp.where(lanes == lane, pv0, zl))
            w1 = jnp.sum(jnp.where(lanes == lane, pv1, zl))
            for cch in range(D // L):
                sl = pl.ds(cch * L, L)
                o_v[t, sl] = w0 * a_v[t, sl] + w1 * b_v[t, sl]
            return 0

        lax.fori_loop(0, GT, comb_body, 0)
        pltpu.sync_copy(o_v, out_hbm.at[pl.ds(t0 + k * GT, GT)])


def _combine_call(ys, d0, d1, p0, p1):
    f32 = jnp.float32
    i32 = jnp.int32
    mesh = plsc.VectorSubcoreMesh(core_axis_name="c", subcore_axis_name="s",
                                  num_cores=NC, num_subcores=NS)
    return pl.kernel(
        _combine_body,
        out_type=jax.ShapeDtypeStruct((N, D), f32),
        mesh=mesh,
        compiler_params=pltpu.CompilerParams(needs_layout_passes=False),
        scratch_types=[
            pltpu.VMEM((TW,), i32), pltpu.VMEM((TW,), i32),
            pltpu.VMEM((TW,), f32), pltpu.VMEM((TW,), f32),
            pltpu.VMEM((GT, D), f32), pltpu.VMEM((GT, D), f32),
            pltpu.VMEM((GT, D), f32),
            pltpu.SemaphoreType.DMA,
        ],
    )(ys, d0, d1, p0, p1)


def kernel(x, Wg, bg, W1, b1, W2, b2):
    p0, p1, e0, e1, r0, r1, starts, te, aux = _gate_call(x, Wg,
                                                         bg.reshape(1, E))
    flat = lambda a: a.reshape(-1)
    xs, d0, d1 = _route_call(starts.reshape(E), flat(e0), flat(e1),
                             flat(r0), flat(r1), x)
    ys = _ffn_call(te.reshape(NTE)[:NT], xs,
                   W1, b1.reshape(E, 1, F), W2, b2.reshape(E, 1, D))
    out = _combine_call(ys, d0, d1, flat(p0), flat(p1))
    return out, aux.reshape(())


# speedup vs baseline: 3.1235x; 1.0105x over previous
import functools

import jax
import jax.numpy as jnp
from jax import lax
from jax.experimental import pallas as pl
from jax.experimental.pallas import tpu as pltpu
from jax.experimental.pallas import tpu_sc as plsc

E = 16
D = 1024
F = 4096
N = 4096
TB = 256
NB = N // TB
TP = 256
PMAX = 2 * N + E * TP
NT = PMAX // TP
NTE = 64
L = 16
NC = 2
NS = 16
NW = NC * NS
QW = PMAX // NW
TW = N // NW
GC = 64
GT = 32


def _gate_body(x_ref, wg_ref, bg_ref,
               p0_ref, p1_ref, e0_ref, e1_ref, r0_ref, r1_ref,
               starts_ref, te_ref, aux_ref,
               carry, tpe, psum):
    b = pl.program_id(0)

    @pl.when(b == 0)
    def _init():
        carry[...] = jnp.zeros_like(carry)
        tpe[...] = jnp.zeros_like(tpe)
        psum[...] = jnp.zeros_like(psum)

    xb = x_ref[...]
    logits = jnp.dot(xb, wg_ref[...], preferred_element_type=jnp.float32)
    logits = logits + bg_ref[...]
    m = jnp.max(logits, axis=1, keepdims=True)
    ex = jnp.exp(logits - m)
    probs = ex / jnp.sum(ex, axis=1, keepdims=True)

    iota = lax.broadcasted_iota(jnp.int32, (TB, E), 1)
    m1 = jnp.max(probs, axis=1, keepdims=True)
    a1 = jnp.min(jnp.where(probs == m1, iota, E), axis=1, keepdims=True)
    p0sel = iota == a1
    masked = jnp.where(p0sel, -jnp.inf, probs)
    m2 = jnp.max(masked, axis=1, keepdims=True)
    a2 = jnp.min(jnp.where(masked == m2, iota, E), axis=1, keepdims=True)
    p1sel = iota == a2

    P0 = p0sel.astype(jnp.float32)
    P1 = p1sel.astype(jnp.float32)
    S = P0 + P1
    ti = lax.broadcasted_iota(jnp.int32, (TB, TB), 0)
    tj = lax.broadcasted_iota(jnp.int32, (TB, TB), 1)
    tril = (ti > tj).astype(jnp.float32)
    excl = jnp.dot(tril, S, preferred_element_type=jnp.float32)
    base = carry[...] + excl
    r0 = jnp.sum(base * P0, axis=1)
    r1 = jnp.sum(base * P1, axis=1)

    p0_ref[...] = m1[:, 0].reshape(1, 1, TB)
    p1_ref[...] = m2[:, 0].reshape(1, 1, TB)
    e0_ref[...] = a1[:, 0].reshape(1, 1, TB)
    e1_ref[...] = a2[:, 0].reshape(1, 1, TB)
    r0_ref[...] = r0.astype(jnp.int32).reshape(1, 1, TB)
    r1_ref[...] = r1.astype(jnp.int32).reshape(1, 1, TB)

    carry[...] = carry[...] + jnp.sum(S, axis=0, keepdims=True)
    tpe[...] = tpe[...] + jnp.sum(P0, axis=0, keepdims=True)
    psum[...] = psum[...] + jnp.sum(probs, axis=0, keepdims=True)

    padded = jnp.floor((carry[...] + (TP - 1)) / TP) * TP
    ei = lax.broadcasted_iota(jnp.int32, (E, E), 0)
    ej = lax.broadcasted_iota(jnp.int32, (E, E), 1)
    incl = (ei <= ej).astype(jnp.float32)
    ends = jnp.dot(padded, incl, preferred_element_type=jnp.float32)
    starts_ref[...] = (ends - padded).astype(jnp.int32)
    gv = lax.broadcasted_iota(jnp.int32, (NTE, 1), 0).astype(jnp.float32) * TP
    te = jnp.sum((gv >= ends).astype(jnp.int32), axis=1)
    te_ref[...] = jnp.minimum(te, E - 1).reshape(1, NTE)
    aux_ref[...] = (jnp.sum(tpe[...] / (N + 1e-8) * (psum[...] / N)) * E
                    ).reshape(1, 1)


def _gate_call(x, Wg, bg2):
    f32 = jnp.float32
    i32 = jnp.int32
    pair_shape = jax.ShapeDtypeStruct((NB, 1, TB), f32)
    pair_ishape = jax.ShapeDtypeStruct((NB, 1, TB), i32)
    pair_spec = pl.BlockSpec((1, 1, TB), lambda i: (i, 0, 0))
    one_spec = lambda s: pl.BlockSpec(s, lambda i: tuple(0 for _ in s))
    return pl.pallas_call(
        _gate_body,
        grid=(NB,),
        in_specs=[
            pl.BlockSpec((TB, D), lambda i: (i, 0)),
            one_spec((D, E)),
            one_spec((1, E)),
        ],
        out_specs=[pair_spec, pair_spec, pair_spec, pair_spec, pair_spec,
                   pair_spec, one_spec((1, E)), one_spec((1, NTE)),
                   one_spec((1, 1))],
        out_shape=[pair_shape, pair_shape, pair_ishape, pair_ishape,
                   pair_ishape, pair_ishape,
                   jax.ShapeDtypeStruct((1, E), i32),
                   jax.ShapeDtypeStruct((1, NTE), i32),
                   jax.ShapeDtypeStruct((1, 1), f32)],
        scratch_shapes=[pltpu.VMEM((1, E), f32)] * 3,
    )(x, Wg, bg2)


KCH = TW // GC


def _route_body(starts_hbm, e0_hbm, e1_hbm, r0_hbm, r1_hbm, x_hbm,
                xs_hbm, d0_hbm, d1_hbm,
                starts_v, e0_v, e1_v, r0_v, r1_v,
                d0f_v, d1f_v, d0i_v, d1i_v, rows_v, sem):
    wid = lax.axis_index("s") * NC + lax.axis_index("c")
    t0 = wid * TW

    pltpu.sync_copy(starts_hbm, starts_v)
    pltpu.sync_copy(e0_hbm.at[pl.ds(t0, TW)], e0_v)
    pltpu.sync_copy(e1_hbm.at[pl.ds(t0, TW)], e1_v)
    pltpu.sync_copy(r0_hbm.at[pl.ds(t0, TW)], r0_v)
    pltpu.sync_copy(r1_hbm.at[pl.ds(t0, TW)], r1_v)

    for k in range(TW // L):
        sl = pl.ds(k * L, L)
        d0c = plsc.load_gather(starts_v, [e0_v[sl]]) + r0_v[sl]
        d1c = plsc.load_gather(starts_v, [e1_v[sl]]) + r1_v[sl]
        d0f_v[sl] = d0c
        d1f_v[sl] = d1c
        d0i_v[k // 4, pl.ds((k % 4) * L, L)] = d0c
        d1i_v[k // 4, pl.ds((k % 4) * L, L)] = d1c
    pltpu.sync_copy(d0f_v, d0_hbm.at[pl.ds(t0, TW)])
    pltpu.sync_copy(d1f_v, d1_hbm.at[pl.ds(t0, TW)])

    for c in range(KCH):
        pltpu.sync_copy(x_hbm.at[pl.ds(t0 + c * GC, GC)], rows_v)
        cp0 = pltpu.async_copy(rows_v, xs_hbm.at[d0i_v.at[c]], sem)
        cp1 = pltpu.async_copy(rows_v, xs_hbm.at[d1i_v.at[c]], sem)
        cp0.wait()
        cp1.wait()


def _route_call(starts, e0, e1, r0, r1, x):
    f32 = jnp.float32
    i32 = jnp.int32
    mesh = plsc.VectorSubcoreMesh(core_axis_name="c", subcore_axis_name="s",
                                  num_cores=NC, num_subcores=NS)
    return pl.kernel(
        _route_body,
        out_type=[jax.ShapeDtypeStruct((PMAX, D), f32),
                  jax.ShapeDtypeStruct((N,), i32),
                  jax.ShapeDtypeStruct((N,), i32)],
        mesh=mesh,
        compiler_params=pltpu.CompilerParams(needs_layout_passes=False),
        scratch_types=[
            pltpu.VMEM((E,), i32),
            pltpu.VMEM((TW,), i32), pltpu.VMEM((TW,), i32),
            pltpu.VMEM((TW,), i32), pltpu.VMEM((TW,), i32),
            pltpu.VMEM((TW,), i32), pltpu.VMEM((TW,), i32),
            pltpu.VMEM((KCH, GC), i32), pltpu.VMEM((KCH, GC), i32),
            pltpu.VMEM((GC, D), f32),
            pltpu.SemaphoreType.DMA,
        ],
    )(starts, e0, e1, r0, r1, x)


def _ffn1_body(te_ref, xs_ref, w1_ref, b1_ref, h_ref):
    h = jnp.dot(xs_ref[...], w1_ref[0], preferred_element_type=jnp.float32)
    h_ref[...] = jnp.maximum(h + b1_ref[0], 0.0).astype(jnp.bfloat16)


def _ffn2_body(te_ref, h_ref, w2_ref, b2_ref, ys_ref):
    w2b = w2_ref[0].astype(jnp.bfloat16)
    y = jnp.dot(h_ref[...], w2b, preferred_element_type=jnp.float32)
    ys_ref[...] = y + b2_ref[0]


def _ffn_call(te, xs, W1, b1, W2, b2):
    g1 = pltpu.PrefetchScalarGridSpec(
        num_scalar_prefetch=1,
        grid=(NT,),
        in_specs=[
            pl.BlockSpec((TP, D), lambda i, te: (i, 0)),
            pl.BlockSpec((1, D, F), lambda i, te: (te[i], 0, 0)),
            pl.BlockSpec((1, 1, F), lambda i, te: (te[i], 0, 0)),
        ],
        out_specs=pl.BlockSpec((TP, F), lambda i, te: (i, 0)),
    )
    h = pl.pallas_call(
        _ffn1_body,
        grid_spec=g1,
        out_shape=jax.ShapeDtypeStruct((PMAX, F), jnp.bfloat16),
        compiler_params=pltpu.CompilerParams(
            dimension_semantics=("arbitrary",)),
    )(te, xs, W1, b1)
    g2 = pltpu.PrefetchScalarGridSpec(
        num_scalar_prefetch=1,
        grid=(NT,),
        in_specs=[
            pl.BlockSpec((TP, F), lambda i, te: (i, 0)),
            pl.BlockSpec((1, F, D), lambda i, te: (te[i], 0, 0)),
            pl.BlockSpec((1, 1, D), lambda i, te: (te[i], 0, 0)),
        ],
        out_specs=pl.BlockSpec((TP, D), lambda i, te: (i, 0)),
    )
    return pl.pallas_call(
        _ffn2_body,
        grid_spec=g2,
        out_shape=jax.ShapeDtypeStruct((PMAX, D), jnp.float32),
        compiler_params=pltpu.CompilerParams(
            dimension_semantics=("arbitrary",)),
    )(te, h, W2, b2)


def _combine_body(ys_hbm, d0_hbm, d1_hbm, p0_hbm, p1_hbm, out_hbm,
                  d0_v, d1_v, p0_v, p1_v, a_v, b_v, o_v, sem):
    wid = lax.axis_index("s") * NC + lax.axis_index("c")
    t0 = wid * TW
    pltpu.sync_copy(d0_hbm.at[pl.ds(t0, TW)], d0_v)
    pltpu.sync_copy(d1_hbm.at[pl.ds(t0, TW)], d1_v)
    pltpu.sync_copy(p0_hbm.at[pl.ds(t0, TW)], p0_v)
    pltpu.sync_copy(p1_hbm.at[pl.ds(t0, TW)], p1_v)

    lanes = lax.iota(jnp.int32, L)
    zl = jnp.zeros((L,), jnp.float32)

    for k in range(TW // GT):
        cpa = pltpu.async_copy(ys_hbm.at[d0_v.at[pl.ds(k * GT, GT)]], a_v,
                               sem)
        cpb = pltpu.async_copy(ys_hbm.at[d1_v.at[pl.ds(k * GT, GT)]], b_v,
                               sem)
        cpa.wait()
        cpb.wait()

        def comb_body(t, _):
            g = k * GT + t
            base = jnp.bitwise_and(g, ~(L - 1))
            lane = jnp.bitwise_and(g, L - 1)
            pv0 = p0_v[pl.ds(base, L)]
            pv1 = p1_v[pl.ds(base, L)]
            w0 = jnp.sum(jnp.where(lanes == lane, pv0, zl))
            w1 = jnp.sum(jnp.where(lanes == lane, pv1, zl))
            for cch in range(D // L):
                sl = pl.ds(cch * L, L)
                o_v[t, sl] = w0 * a_v[t, sl] + w1 * b_v[t, sl]
            return 0

        lax.fori_loop(0, GT, comb_body, 0)
        pltpu.sync_copy(o_v, out_hbm.at[pl.ds(t0 + k * GT, GT)])


def _combine_call(ys, d0, d1, p0, p1):
    f32 = jnp.float32
    i32 = jnp.int32
    mesh = plsc.VectorSubcoreMesh(core_axis_name="c", subcore_axis_name="s",
                                  num_cores=NC, num_subcores=NS)
    return pl.kernel(
        _combine_body,
        out_type=jax.ShapeDtypeStruct((N, D), f32),
        mesh=mesh,
        compiler_params=pltpu.CompilerParams(needs_layout_passes=False),
        scratch_types=[
            pltpu.VMEM((TW,), i32), pltpu.VMEM((TW,), i32),
            pltpu.VMEM((TW,), f32), pltpu.VMEM((TW,), f32),
            pltpu.VMEM((GT, D), f32), pltpu.VMEM((GT, D), f32),
            pltpu.VMEM((GT, D), f32),
            pltpu.SemaphoreType.DMA,
        ],
    )(ys, d0, d1, p0, p1)


def kernel(x, Wg, bg, W1, b1, W2, b2):
    p0, p1, e0, e1, r0, r1, starts, te, aux = _gate_call(x, Wg,
                                                         bg.reshape(1, E))
    flat = lambda a: a.reshape(-1)
    xs, d0, d1 = _route_call(starts.reshape(E), flat(e0), flat(e1),
                             flat(r0), flat(r1), x)
    ys = _ffn_call(te.reshape(NTE)[:NT], xs,
                   W1, b1.reshape(E, 1, F), W2, b2.reshape(E, 1, D))
    out = _combine_call(ys, d0, d1, flat(p0), flat(p1))
    return out, aux.reshape(())
